# Initial kernel scaffold; baseline (speedup 1.0000x reference)
import jax
import jax.numpy as jnp
from jax.experimental import pallas as pl


def kernel(text_ids, table):
    raise NotImplementedError("write your pallas kernel here")


# speedup vs baseline: 8.5882x; 8.5882x over previous
import functools

import jax
import jax.numpy as jnp
from jax import lax
from jax.experimental import pallas as pl
from jax.experimental.pallas import tpu as pltpu
from jax.experimental.pallas import tpu_sc as plsc

B = 4096
L = 50
D = 128
LANES = 16
NVREG = D // LANES
NB = 8


@functools.cache
def _build():
    info = plsc.get_sparse_core_info()
    nw = info.num_cores * info.num_subcores
    b_per_w = B // nw
    n_chunks = b_per_w // NB
    mesh = plsc.VectorSubcoreMesh(core_axis_name="c", subcore_axis_name="s")

    @functools.partial(
        pl.kernel,
        mesh=mesh,
        out_type=jax.ShapeDtypeStruct((B, D), jnp.float32),
        scratch_types=[
            pltpu.VMEM((NB * L,), jnp.int32),
            pltpu.VMEM((NB * L, D), jnp.float32),
            pltpu.VMEM((NB, D), jnp.float32),
            pltpu.SemaphoreType.DMA,
        ],
    )
    def k(ids_hbm, table_hbm, out_hbm, idx_v, rows_v, out_v, sem):
        wid = lax.axis_index("s") * info.num_cores + lax.axis_index("c")
        base_row = wid * b_per_w

        @pl.loop(0, n_chunks)
        def _chunk(c):
            row0 = base_row + c * NB
            pltpu.sync_copy(ids_hbm.at[pl.ds(row0 * L, NB * L)], idx_v)
            pltpu.async_copy(table_hbm.at[idx_v], rows_v, sem).wait()
            for r in range(NB):
                zeros = tuple(
                    jnp.zeros((LANES,), jnp.float32) for _ in range(NVREG)
                )

                def acc_fn(l, acc, r=r):
                    return tuple(
                        acc[j] + rows_v[r * L + l, pl.ds(j * LANES, LANES)]
                        for j in range(NVREG)
                    )
                acc = pl.loop(0, L, init_carry=zeros, unroll=5)(acc_fn)
                for j in range(NVREG):
                    out_v[r, pl.ds(j * LANES, LANES)] = acc[j] * (1.0 / L)
            pltpu.sync_copy(out_v, out_hbm.at[pl.ds(row0, NB)])

    return k


def kernel(text_ids, table):
    ids_flat = text_ids.reshape(-1).astype(jnp.int32)
    return _build()(ids_flat, table)


# speedup vs baseline: 13.4194x; 1.5625x over previous
import functools

import jax
import jax.numpy as jnp
from jax import lax
from jax.experimental import pallas as pl
from jax.experimental.pallas import tpu as pltpu
from jax.experimental.pallas import tpu_sc as plsc

B = 4096
L = 50
D = 128
LANES = 16
NVREG = D // LANES
NB = 8


@functools.cache
def _build():
    info = plsc.get_sparse_core_info()
    nw = info.num_cores * info.num_subcores
    b_per_w = B // nw
    n_chunks = b_per_w // NB
    assert n_chunks % 2 == 0
    mesh = plsc.VectorSubcoreMesh(core_axis_name="c", subcore_axis_name="s")

    @functools.partial(
        pl.kernel,
        mesh=mesh,
        out_type=jax.ShapeDtypeStruct((B, D), jnp.float32),
        scratch_types=[
            pltpu.VMEM((b_per_w * L,), jnp.int32),
            pltpu.VMEM((2, NB * L, D), jnp.float32),
            pltpu.VMEM((NB, D), jnp.float32),
            [pltpu.SemaphoreType.DMA, pltpu.SemaphoreType.DMA],
        ],
    )
    def k(ids_hbm, table_hbm, out_hbm, idx_v, rows_v, out_v, sems):
        wid = lax.axis_index("s") * info.num_cores + lax.axis_index("c")
        base_row = wid * b_per_w

        pltpu.sync_copy(ids_hbm.at[pl.ds(base_row * L, b_per_w * L)], idx_v)

        def gather(c, buf):
            return pltpu.make_async_copy(
                table_hbm.at[idx_v.at[pl.ds(c * (NB * L), NB * L)]],
                rows_v.at[buf],
                sems[buf],
            )

        def compute(c, buf):
            for r in range(NB):
                zeros = tuple(
                    jnp.zeros((LANES,), jnp.float32) for _ in range(NVREG)
                )

                def acc_fn(l, acc, r=r):
                    return tuple(
                        acc[j] + rows_v[buf, r * L + l, pl.ds(j * LANES, LANES)]
                        for j in range(NVREG)
                    )
                acc = pl.loop(0, L, init_carry=zeros, unroll=5)(acc_fn)
                for j in range(NVREG):
                    out_v[r, pl.ds(j * LANES, LANES)] = acc[j] * (1.0 / L)
            pltpu.sync_copy(out_v, out_hbm.at[pl.ds(base_row + c * NB, NB)])

        gather(0, 0).start()

        @pl.loop(0, n_chunks, step=2)
        def _chunks(g):
            for buf in range(2):
                c = g + buf
                nxt = c + 1

                @pl.when(nxt < n_chunks)
                def _():
                    gather(nxt, 1 - buf).start()

                gather(c, buf).wait()
                compute(c, buf)

    return k


def kernel(text_ids, table):
    ids_flat = text_ids.reshape(-1).astype(jnp.int32)
    return _build()(ids_flat, table)
